# Initial kernel scaffold; baseline (speedup 1.0000x reference)
import jax
import jax.numpy as jnp
from jax.experimental import pallas as pl


def kernel(rec, data):
    raise NotImplementedError("write your pallas kernel here")


# speedup vs baseline: 1.3832x; 1.3832x over previous
import jax
import jax.numpy as jnp
from jax.experimental import pallas as pl

_B, _N, _M = 8, 2048, 2048
_NT = 4
_NB = _N // _NT


def _chamfer_body(x_ref, yt_ref, dx_ref, dy_ref):
    x = x_ref[0]
    yt = yt_ref[0]
    d0 = x[:, 0:1] - yt[0:1, :]
    d1 = x[:, 1:2] - yt[1:2, :]
    d2 = x[:, 2:3] - yt[2:3, :]
    d = d0 * d0 + d1 * d1 + d2 * d2
    dx_ref[0, 0] = jnp.min(d, axis=1)
    colmin = jnp.min(d, axis=0)
    n = pl.program_id(1)

    @pl.when(n == 0)
    def _():
        dy_ref[0, 0] = colmin

    @pl.when(n != 0)
    def _():
        dy_ref[0, 0] = jnp.minimum(dy_ref[0, 0], colmin)


def kernel(rec, data):
    dataT = jnp.transpose(data, (0, 2, 1))
    dist_x, dist_y = pl.pallas_call(
        _chamfer_body,
        grid=(_B, _NT),
        in_specs=[
            pl.BlockSpec((1, _NB, 3), lambda b, n: (b, n, 0)),
            pl.BlockSpec((1, 3, _M), lambda b, n: (b, 0, 0)),
        ],
        out_specs=[
            pl.BlockSpec((1, 1, _NB), lambda b, n: (b * _NT + n, 0, 0)),
            pl.BlockSpec((1, 1, _M), lambda b, n: (b, 0, 0)),
        ],
        out_shape=[
            jax.ShapeDtypeStruct((_B * _NT, 1, _NB), jnp.float32),
            jax.ShapeDtypeStruct((_B, 1, _M), jnp.float32),
        ],
    )(rec, dataT)
    dist_x = dist_x.reshape(_B, _N)
    dist_y = dist_y.reshape(_B, _M)
    per_batch = jnp.maximum(jnp.mean(dist_y, axis=1), jnp.mean(dist_x, axis=1))
    return jnp.mean(per_batch)
